# Initial kernel scaffold; baseline (speedup 1.0000x reference)
import jax
import jax.numpy as jnp
from jax.experimental import pallas as pl


def kernel(x, edge_index, W_conv, b_conv, W1, b1, W2, b2):
    raise NotImplementedError("write your pallas kernel here")


# speedup vs baseline: 3.0121x; 3.0121x over previous
import jax
import jax.numpy as jnp
from jax import lax
from jax.experimental import pallas as pl
from jax.experimental.pallas import tpu as pltpu
from jax.experimental.pallas import tpu_sc as plsc

_N = 10000
_NP = 10240
_NH = _NP // 2
_E = 320000
_D = 128
_K = 16
_ALPHA = 0.6
_CK = (1.0 - _ALPHA) / _K
_NS = 16
_L = 16
_NQ = _D // _L
_CB = 128
_CH = 160
_GP = 8
_NG = _CH // _GP
_RPC = _NH // _NS
_AC = 64
_NB = _RPC // _AC
_DUMP = _NH


def _scratch_types():
    return [
        pltpu.VMEM((_GP, _CB), jnp.int32),
        pltpu.VMEM((_GP, _CB), jnp.int32),
        pltpu.VMEM((_GP, _CB), jnp.int32),
        pltpu.VMEM((2, _CB, _D), jnp.float32),
        pltpu.VMEM((_AC, _D), jnp.float32),
        pltpu.VMEM_SHARED((_NH + 8, _D), jnp.float32),
    ] + [pltpu.SemaphoreType.DMA] * 4


def _unpack_group(pbuf, src_i, dst_i, cNH, with_src):
    for b in range(_GP):
        for j in range(_CB // _L):
            sl = pl.ds(j * _L, _L)
            v = pbuf[b, sl]
            d2 = (v >> 16) - cNH
            ok = (d2 >= 0) & (d2 < _NH)
            dst_i[b, sl] = jnp.where(ok, d2, _DUMP)
            if with_src:
                src_i[b, sl] = v & 0xFFFF


def _init_body(xp, pidx, z0, dinv, sqc,
               pbuf, src_i, dst_i, gbuf, abuf, acc, g0, g1, s0, s1):
    c = lax.axis_index("c")
    s = lax.axis_index("s")
    cNH = c * _NH
    gbase = cNH + s * _RPC
    abase = s * _RPC
    ssems = (s0, s1)

    def _zrow(i, _):
        for q in range(_NQ):
            abuf[i, pl.ds(q * _L, _L)] = jnp.zeros((_L,), jnp.float32)
        return 0
    lax.fori_loop(0, _AC, _zrow, 0)
    for k in range(_NB):
        pltpu.sync_copy(abuf, acc.at[pl.ds(abase + k * _AC, _AC)])
    plsc.subcore_barrier()

    def _orow(i, _):
        for q in range(_NQ):
            gbuf[0, i, pl.ds(q * _L, _L)] = jnp.ones((_L,), jnp.float32)
        return 0
    lax.fori_loop(0, _CB, _orow, 0)

    def _dgroup(g, _):
        pltpu.sync_copy(pidx.at[s, pl.ds(g * _GP, _GP)], pbuf)
        _unpack_group(pbuf, src_i, dst_i, cNH, with_src=False)
        for b in range(_GP):
            r = b % 2
            pltpu.async_copy(gbuf.at[0], acc.at[dst_i.at[b]],
                             ssems[r], add=True)
            pltpu.make_async_copy(gbuf.at[0], acc.at[dst_i.at[b]],
                                  ssems[r]).wait()
        return 0
    lax.fori_loop(0, _NG, _dgroup, 0)
    plsc.subcore_barrier()

    for k in range(_NB):
        goff = gbase + k * _AC
        pltpu.sync_copy(acc.at[pl.ds(abase + k * _AC, _AC)], abuf)
        pltpu.sync_copy(xp.at[pl.ds(goff, _AC)], gbuf.at[1, pl.ds(0, _AC)])

        def _row(i, _):
            d = abuf[i, pl.ds(0, _L)] + 1.0
            y = 0.5 * (d + 1.0)
            for _it in range(18):
                y = 0.5 * (y + d / y)
            dis = 1.0 / y
            dnv = 1.0 / d
            sq = y * _CK
            for q in range(_NQ):
                sl = pl.ds(q * _L, _L)
                abuf[i, sl] = dnv
                gbuf[0, i, sl] = gbuf[1, i, sl] * dis
                gbuf[1, i, sl] = sq
            return 0
        lax.fori_loop(0, _AC, _row, 0)
        pltpu.sync_copy(gbuf.at[0, pl.ds(0, _AC)], z0.at[pl.ds(goff, _AC)])
        pltpu.sync_copy(abuf, dinv.at[pl.ds(goff, _AC)])
        pltpu.sync_copy(gbuf.at[1, pl.ds(0, _AC)], sqc.at[pl.ds(goff, _AC)])


def _round_body(pidx, z_in, zsum_in, dinv, z_out, zsum_out,
                pbuf, src_i, dst_i, gbuf, abuf, acc, g0, g1, s0, s1):
    c = lax.axis_index("c")
    s = lax.axis_index("s")
    cNH = c * _NH
    gbase = cNH + s * _RPC
    abase = s * _RPC
    gsems = (g0, g1)
    ssems = (s0, s1)

    for k in range(_NB):
        pltpu.sync_copy(z_in.at[pl.ds(gbase + k * _AC, _AC)], abuf)
        pltpu.sync_copy(abuf, acc.at[pl.ds(abase + k * _AC, _AC)])
    plsc.subcore_barrier()

    def _group(g, _):
        pltpu.sync_copy(pidx.at[s, pl.ds(g * _GP, _GP)], pbuf)
        _unpack_group(pbuf, src_i, dst_i, cNH, with_src=True)
        pltpu.async_copy(z_in.at[src_i.at[0]], gbuf.at[0], gsems[0])
        for b in range(_GP):
            r = b % 2
            pltpu.make_async_copy(z_in.at[src_i.at[b]], gbuf.at[r],
                                  gsems[r]).wait()
            if b + 1 < _GP:
                pltpu.async_copy(z_in.at[src_i.at[b + 1]],
                                 gbuf.at[1 - r], gsems[1 - r])
            pltpu.async_copy(gbuf.at[r], acc.at[dst_i.at[b]],
                             ssems[r], add=True)
            pltpu.make_async_copy(gbuf.at[r], acc.at[dst_i.at[b]],
                                  ssems[r]).wait()
        return 0
    lax.fori_loop(0, _NG, _group, 0)
    plsc.subcore_barrier()

    for k in range(_NB):
        goff = gbase + k * _AC
        pltpu.sync_copy(acc.at[pl.ds(abase + k * _AC, _AC)], abuf)
        pltpu.sync_copy(dinv.at[pl.ds(goff, _AC)], gbuf.at[0, pl.ds(0, _AC)])
        pltpu.sync_copy(zsum_in.at[pl.ds(goff, _AC)],
                        gbuf.at[1, pl.ds(0, _AC)])

        def _row(i, _):
            for q in range(_NQ):
                sl = pl.ds(q * _L, _L)
                zv = abuf[i, sl] * gbuf[0, i, sl]
                abuf[i, sl] = zv
                gbuf[1, i, sl] = gbuf[1, i, sl] + zv
            return 0
        lax.fori_loop(0, _AC, _row, 0)
        pltpu.sync_copy(abuf, z_out.at[pl.ds(goff, _AC)])
        pltpu.sync_copy(gbuf.at[1, pl.ds(0, _AC)],
                        zsum_out.at[pl.ds(goff, _AC)])


def _sc_mesh():
    return plsc.VectorSubcoreMesh(core_axis_name="c", subcore_axis_name="s")


def _init_sc(xp, pidx):
    f = pl.kernel(
        _init_body,
        out_type=[jax.ShapeDtypeStruct((_NP, _D), jnp.float32)] * 3,
        mesh=_sc_mesh(),
        scratch_types=_scratch_types(),
    )
    return f(xp, pidx)


def _round_sc(pidx, z, zsum, dinv):
    f = pl.kernel(
        _round_body,
        out_type=[jax.ShapeDtypeStruct((_NP, _D), jnp.float32)] * 2,
        mesh=_sc_mesh(),
        scratch_types=_scratch_types(),
    )
    return f(pidx, z, zsum, dinv)


_MB = 1024


def _mlp_body(x_ref, zs_ref, sq_ref, wc_ref, bc_ref, w1_ref, b1_ref,
              w2_ref, b2_ref, o_ref):
    h = _ALPHA * x_ref[...] + zs_ref[...] * sq_ref[...]
    y = jnp.dot(h, wc_ref[...], preferred_element_type=jnp.float32)
    y = jnp.maximum(y + bc_ref[...], 0.0)
    y = jnp.dot(y, w1_ref[...], preferred_element_type=jnp.float32)
    y = jnp.maximum(y + b1_ref[...], 0.0)
    y = jnp.dot(y, w2_ref[...], preferred_element_type=jnp.float32)
    o_ref[...] = y + b2_ref[...]


def _mlp(xp, zsum, sqc, wc, bc, w1, b1, w2, b2):
    bspec = pl.BlockSpec((_MB, _D), lambda i: (i, 0))
    wspec = pl.BlockSpec((_D, _D), lambda i: (0, 0))
    vspec = pl.BlockSpec((1, _D), lambda i: (0, 0))
    return pl.pallas_call(
        _mlp_body,
        grid=(_NP // _MB,),
        in_specs=[bspec, bspec, bspec,
                  wspec, vspec, wspec, vspec, wspec, vspec],
        out_specs=bspec,
        out_shape=jax.ShapeDtypeStruct((_NP, _D), jnp.float32),
    )(xp, zsum, sqc, wc, bc.reshape(1, _D), w1, b1.reshape(1, _D),
      w2, b2.reshape(1, _D))


def kernel(x, edge_index, W_conv, b_conv, W1, b1, W2, b2):
    xp = jnp.pad(x, ((0, _NP - _N), (0, 0)))
    src = edge_index[0]
    dst = edge_index[1]
    packed = src | (dst << 16)
    padn = _NS * _CH * _CB - _E
    packed = jnp.concatenate(
        [packed, jnp.full((padn,), _NP << 16, jnp.int32)])
    pidx = packed.reshape(_NS, _CH, _CB)

    z, dinv, sqc = _init_sc(xp, pidx)
    zsum = jnp.zeros((_NP, _D), jnp.float32)
    for _t in range(_K):
        z, zsum = _round_sc(pidx, z, zsum, dinv)
    out = _mlp(xp, zsum, sqc, W_conv, b_conv, W1, b1, W2, b2)
    return out[:_N]


# speedup vs baseline: 3.0330x; 1.0069x over previous
import jax
import jax.numpy as jnp
from jax import lax
from jax.experimental import pallas as pl
from jax.experimental.pallas import tpu as pltpu
from jax.experimental.pallas import tpu_sc as plsc

_N = 10000
_NP = 10240
_NH = _NP // 2
_E = 320000
_D = 128
_K = 16
_ALPHA = 0.6
_CK = (1.0 - _ALPHA) / _K
_NS = 16
_L = 16
_NQ = _D // _L
_CB = 128
_CH = 160
_GP = 16
_NG = _CH // _GP
_SS = 2
_SU = _GP // _SS
_RPC = _NH // _NS
_AC = 64
_NB = _RPC // _AC
_DUMP = _NH


def _scratch_types():
    return [
        pltpu.VMEM((2, _GP, _CB), jnp.int32),
        pltpu.VMEM((_GP * _CB,), jnp.int32),
        pltpu.VMEM((_GP * _CB,), jnp.int32),
        pltpu.VMEM((2, _SS * _CB, _D), jnp.float32),
        pltpu.VMEM((_AC, _D), jnp.float32),
        pltpu.VMEM_SHARED((_NH + 8, _D), jnp.float32),
    ] + [pltpu.SemaphoreType.DMA] * 6


def _unpack_group(pbuf, par, src_i, dst_i, cNH, with_src):
    for b in range(_GP):
        for j in range(_CB // _L):
            sl = pl.ds(b * _CB + j * _L, _L)
            v = pbuf[par, b, pl.ds(j * _L, _L)]
            d2 = (v >> 16) - cNH
            ok = (d2 >= 0) & (d2 < _NH)
            dst_i[sl] = jnp.where(ok, d2, _DUMP)
            if with_src:
                src_i[sl] = v & 0xFFFF


def _pipeline(z_in, acc, src_i, dst_i, gbuf, gsems, ssems):
    def _si(u):
        return src_i.at[pl.ds(_SS * _CB * u, _SS * _CB)]

    def _di(u):
        return dst_i.at[pl.ds(_SS * _CB * u, _SS * _CB)]

    pltpu.async_copy(z_in.at[_si(0)], gbuf.at[0], gsems[0])
    for u in range(_SU):
        r = u % 2
        pltpu.make_async_copy(z_in.at[_si(u)], gbuf.at[r], gsems[r]).wait()
        pltpu.async_copy(gbuf.at[r], acc.at[_di(u)], ssems[r], add=True)
        if u + 1 < _SU:
            if u >= 1:
                pltpu.make_async_copy(gbuf.at[1 - r], acc.at[_di(u - 1)],
                                      ssems[1 - r]).wait()
            pltpu.async_copy(z_in.at[_si(u + 1)], gbuf.at[1 - r],
                             gsems[1 - r])
    pltpu.make_async_copy(gbuf.at[0], acc.at[_di(_SU - 2)], ssems[0]).wait()
    pltpu.make_async_copy(gbuf.at[1], acc.at[_di(_SU - 1)], ssems[1]).wait()


def _init_body(xp, pidx, z0, dinv, sqc,
               pbuf, src_i, dst_i, gbuf, abuf, acc, p0, p1, g0, g1, s0, s1):
    c = lax.axis_index("c")
    s = lax.axis_index("s")
    cNH = c * _NH
    gbase = cNH + s * _RPC
    abase = s * _RPC
    ssems = (s0, s1)

    def _zrow(i, _):
        for q in range(_NQ):
            abuf[i, pl.ds(q * _L, _L)] = jnp.zeros((_L,), jnp.float32)
        return 0
    lax.fori_loop(0, _AC, _zrow, 0)
    for k in range(_NB):
        pltpu.sync_copy(abuf, acc.at[pl.ds(abase + k * _AC, _AC)])
    plsc.subcore_barrier()

    def _orow(i, _):
        for q in range(_NQ):
            gbuf[0, i, pl.ds(q * _L, _L)] = jnp.ones((_L,), jnp.float32)
        return 0
    lax.fori_loop(0, _CB, _orow, 0)

    def _dgroup(g, _):
        pltpu.sync_copy(pidx.at[s, pl.ds(g * _GP, _GP)], pbuf.at[0])
        _unpack_group(pbuf, 0, src_i, dst_i, cNH, with_src=False)
        for b in range(_GP):
            r = b % 2
            ones = gbuf.at[0, pl.ds(0, _CB)]
            di = dst_i.at[pl.ds(b * _CB, _CB)]
            pltpu.async_copy(ones, acc.at[di], ssems[r], add=True)
            pltpu.make_async_copy(ones, acc.at[di], ssems[r]).wait()
        return 0
    lax.fori_loop(0, _NG, _dgroup, 0)
    plsc.subcore_barrier()

    for k in range(_NB):
        goff = gbase + k * _AC
        pltpu.sync_copy(acc.at[pl.ds(abase + k * _AC, _AC)], abuf)
        pltpu.sync_copy(xp.at[pl.ds(goff, _AC)], gbuf.at[1, pl.ds(0, _AC)])

        def _row(i, _):
            d = abuf[i, pl.ds(0, _L)] + 1.0
            y = 0.5 * (d + 1.0)
            for _it in range(18):
                y = 0.5 * (y + d / y)
            dis = 1.0 / y
            dnv = 1.0 / d
            sq = y * _CK
            for q in range(_NQ):
                sl = pl.ds(q * _L, _L)
                abuf[i, sl] = dnv
                gbuf[0, i, sl] = gbuf[1, i, sl] * dis
                gbuf[1, i, sl] = sq
            return 0
        lax.fori_loop(0, _AC, _row, 0)
        pltpu.sync_copy(gbuf.at[0, pl.ds(0, _AC)],
                        z0.at[pl.ds(goff, _AC)])
        pltpu.sync_copy(abuf, dinv.at[pl.ds(goff, _AC)])
        pltpu.sync_copy(gbuf.at[1, pl.ds(0, _AC)],
                        sqc.at[pl.ds(goff, _AC)])


def _round_body(pidx, z_in, zsum_in, dinv, z_out, zsum_out,
                pbuf, src_i, dst_i, gbuf, abuf, acc, p0, p1, g0, g1, s0, s1):
    c = lax.axis_index("c")
    s = lax.axis_index("s")
    cNH = c * _NH
    gbase = cNH + s * _RPC
    abase = s * _RPC
    gsems = (g0, g1)
    ssems = (s0, s1)

    for k in range(_NB):
        pltpu.sync_copy(z_in.at[pl.ds(gbase + k * _AC, _AC)], abuf)
        pltpu.sync_copy(abuf, acc.at[pl.ds(abase + k * _AC, _AC)])
    plsc.subcore_barrier()

    pltpu.async_copy(pidx.at[s, pl.ds(0, _GP)], pbuf.at[0], p0)

    def _pair(i, _):
        ge = 2 * i
        pltpu.make_async_copy(pidx.at[s, pl.ds(ge * _GP, _GP)],
                              pbuf.at[0], p0).wait()
        _unpack_group(pbuf, 0, src_i, dst_i, cNH, with_src=True)
        pltpu.async_copy(pidx.at[s, pl.ds((ge + 1) * _GP, _GP)],
                         pbuf.at[1], p1)
        _pipeline(z_in, acc, src_i, dst_i, gbuf, gsems, ssems)

        pltpu.make_async_copy(pidx.at[s, pl.ds((ge + 1) * _GP, _GP)],
                              pbuf.at[1], p1).wait()
        _unpack_group(pbuf, 1, src_i, dst_i, cNH, with_src=True)

        @pl.when(i + 1 < _NG // 2)
        def _():
            pltpu.async_copy(pidx.at[s, pl.ds((ge + 2) * _GP, _GP)],
                             pbuf.at[0], p0)
        _pipeline(z_in, acc, src_i, dst_i, gbuf, gsems, ssems)
        return 0
    lax.fori_loop(0, _NG // 2, _pair, 0)
    plsc.subcore_barrier()

    for k in range(_NB):
        goff = gbase + k * _AC
        pltpu.sync_copy(acc.at[pl.ds(abase + k * _AC, _AC)], abuf)
        pltpu.sync_copy(dinv.at[pl.ds(goff, _AC)],
                        gbuf.at[0, pl.ds(0, _AC)])
        pltpu.sync_copy(zsum_in.at[pl.ds(goff, _AC)],
                        gbuf.at[1, pl.ds(0, _AC)])

        def _row(i, _):
            for q in range(_NQ):
                sl = pl.ds(q * _L, _L)
                zv = abuf[i, sl] * gbuf[0, i, sl]
                abuf[i, sl] = zv
                gbuf[1, i, sl] = gbuf[1, i, sl] + zv
            return 0
        lax.fori_loop(0, _AC, _row, 0)
        pltpu.sync_copy(abuf, z_out.at[pl.ds(goff, _AC)])
        pltpu.sync_copy(gbuf.at[1, pl.ds(0, _AC)],
                        zsum_out.at[pl.ds(goff, _AC)])


def _sc_mesh():
    return plsc.VectorSubcoreMesh(core_axis_name="c", subcore_axis_name="s")


def _init_sc(xp, pidx):
    f = pl.kernel(
        _init_body,
        out_type=[jax.ShapeDtypeStruct((_NP, _D), jnp.float32)] * 3,
        mesh=_sc_mesh(),
        scratch_types=_scratch_types(),
    )
    return f(xp, pidx)


def _round_sc(pidx, z, zsum, dinv):
    f = pl.kernel(
        _round_body,
        out_type=[jax.ShapeDtypeStruct((_NP, _D), jnp.float32)] * 2,
        mesh=_sc_mesh(),
        scratch_types=_scratch_types(),
    )
    return f(pidx, z, zsum, dinv)


_MB = 1024


def _mlp_body(x_ref, zs_ref, sq_ref, wc_ref, bc_ref, w1_ref, b1_ref,
              w2_ref, b2_ref, o_ref):
    h = _ALPHA * x_ref[...] + zs_ref[...] * sq_ref[...]
    y = jnp.dot(h, wc_ref[...], preferred_element_type=jnp.float32)
    y = jnp.maximum(y + bc_ref[...], 0.0)
    y = jnp.dot(y, w1_ref[...], preferred_element_type=jnp.float32)
    y = jnp.maximum(y + b1_ref[...], 0.0)
    y = jnp.dot(y, w2_ref[...], preferred_element_type=jnp.float32)
    o_ref[...] = y + b2_ref[...]


def _mlp(xp, zsum, sqc, wc, bc, w1, b1, w2, b2):
    bspec = pl.BlockSpec((_MB, _D), lambda i: (i, 0))
    wspec = pl.BlockSpec((_D, _D), lambda i: (0, 0))
    vspec = pl.BlockSpec((1, _D), lambda i: (0, 0))
    return pl.pallas_call(
        _mlp_body,
        grid=(_NP // _MB,),
        in_specs=[bspec, bspec, bspec,
                  wspec, vspec, wspec, vspec, wspec, vspec],
        out_specs=bspec,
        out_shape=jax.ShapeDtypeStruct((_NP, _D), jnp.float32),
    )(xp, zsum, sqc, wc, bc.reshape(1, _D), w1, b1.reshape(1, _D),
      w2, b2.reshape(1, _D))


def kernel(x, edge_index, W_conv, b_conv, W1, b1, W2, b2):
    xp = jnp.pad(x, ((0, _NP - _N), (0, 0)))
    src = edge_index[0]
    dst = edge_index[1]
    packed = src | (dst << 16)
    padn = _NS * _CH * _CB - _E
    packed = jnp.concatenate(
        [packed, jnp.full((padn,), _NP << 16, jnp.int32)])
    pidx = packed.reshape(_NS, _CH, _CB)

    z, dinv, sqc = _init_sc(xp, pidx)
    zsum = jnp.zeros((_NP, _D), jnp.float32)
    for _t in range(_K):
        z, zsum = _round_sc(pidx, z, zsum, dinv)
    out = _mlp(xp, zsum, sqc, W_conv, b_conv, W1, b1, W2, b2)
    return out[:_N]


# speedup vs baseline: 4.2393x; 1.3977x over previous
import jax
import jax.numpy as jnp
from jax import lax
from jax.experimental import pallas as pl
from jax.experimental.pallas import tpu as pltpu
from jax.experimental.pallas import tpu_sc as plsc

_N = 10000
_NP = 10240
_NH = _NP // 2
_E = 320000
_D = 128
_K = 16
_ALPHA = 0.6
_CK = (1.0 - _ALPHA) / _K
_NS = 16
_L = 16
_NQ = _D // _L
_CB = 128
_CH = 160
_GP = 16
_NG = _CH // _GP
_SS = 2
_SU = _GP // _SS
_GE = _GP * _CB
_NGT = _NS * _NG
_RPC = _NH // _NS
_AC = 64
_NB = _RPC // _AC
_DUMP = _NH


def _scratch_types():
    return [
        pltpu.VMEM((2, _GP * _CB), jnp.int32),
        pltpu.VMEM((_GP * _CB,), jnp.int32),
        pltpu.VMEM((_GP * _CB,), jnp.int32),
        pltpu.VMEM((_L,), jnp.int32),
        pltpu.VMEM((2, _SS * _CB, _D), jnp.float32),
        pltpu.VMEM((_AC, _D), jnp.float32),
        pltpu.VMEM_SHARED((_NH + 8, _D), jnp.float32),
    ] + [pltpu.SemaphoreType.DMA] * 6


def _unpack_group(pbuf, par, src_i, dst_i, cNH, with_src):
    for b in range(_GP):
        for j in range(_CB // _L):
            sl = pl.ds(b * _CB + j * _L, _L)
            v = pbuf[par, pl.ds(b * _CB + j * _L, _L)]
            d2 = (v >> 16) - cNH
            ok = (d2 >= 0) & (d2 < _NH)
            dst_i[sl] = jnp.where(ok, d2, _DUMP)
            if with_src:
                src_i[sl] = v & 0xFFFF


def _pipeline(z_in, acc, src_i, dst_i, gbuf, gsems, ssems):
    def _si(u):
        return src_i.at[pl.ds(_SS * _CB * u, _SS * _CB)]

    def _di(u):
        return dst_i.at[pl.ds(_SS * _CB * u, _SS * _CB)]

    pltpu.async_copy(z_in.at[_si(0)], gbuf.at[0], gsems[0])
    for u in range(_SU):
        r = u % 2
        pltpu.make_async_copy(z_in.at[_si(u)], gbuf.at[r], gsems[r]).wait()
        pltpu.async_copy(gbuf.at[r], acc.at[_di(u)], ssems[r], add=True)
        if u + 1 < _SU:
            if u >= 1:
                pltpu.make_async_copy(gbuf.at[1 - r], acc.at[_di(u - 1)],
                                      ssems[1 - r]).wait()
            pltpu.async_copy(z_in.at[_si(u + 1)], gbuf.at[1 - r],
                             gsems[1 - r])
    pltpu.make_async_copy(gbuf.at[0], acc.at[_di(_SU - 2)], ssems[0]).wait()
    pltpu.make_async_copy(gbuf.at[1], acc.at[_di(_SU - 1)], ssems[1]).wait()


def _init_body(xp, pidx, z0, dinv, sqc,
               pbuf, src_i, dst_i, nbuf, gbuf, abuf, acc,
               p0, p1, g0, g1, s0, s1):
    c = lax.axis_index("c")
    s = lax.axis_index("s")
    cNH = c * _NH
    gbase = cNH + s * _RPC
    abase = s * _RPC
    ssems = (s0, s1)

    def _zrow(i, _):
        for q in range(_NQ):
            abuf[i, pl.ds(q * _L, _L)] = jnp.zeros((_L,), jnp.float32)
        return 0
    lax.fori_loop(0, _AC, _zrow, 0)
    for k in range(_NB):
        pltpu.sync_copy(abuf, acc.at[pl.ds(abase + k * _AC, _AC)])
    plsc.subcore_barrier()

    def _orow(i, _):
        for q in range(_NQ):
            gbuf[0, i, pl.ds(q * _L, _L)] = jnp.ones((_L,), jnp.float32)
        return 0
    lax.fori_loop(0, _CB, _orow, 0)

    def _dgroup(g, _):
        pltpu.sync_copy(pidx.at[pl.ds(g * _GE, _GE)], pbuf.at[0])
        _unpack_group(pbuf, 0, src_i, dst_i, cNH, with_src=False)
        for b in range(_GP):
            r = b % 2
            ones = gbuf.at[0, pl.ds(0, _CB)]
            di = dst_i.at[pl.ds(b * _CB, _CB)]
            pltpu.async_copy(ones, acc.at[di], ssems[r], add=True)
            pltpu.make_async_copy(ones, acc.at[di], ssems[r]).wait()
        return 0
    lax.fori_loop(s * _NG, (s + 1) * _NG, _dgroup, 0)
    plsc.subcore_barrier()

    for k in range(_NB):
        goff = gbase + k * _AC
        pltpu.sync_copy(acc.at[pl.ds(abase + k * _AC, _AC)], abuf)
        pltpu.sync_copy(xp.at[pl.ds(goff, _AC)], gbuf.at[1, pl.ds(0, _AC)])

        def _row(i, _):
            d = abuf[i, pl.ds(0, _L)] + 1.0
            y = 0.5 * (d + 1.0)
            for _it in range(18):
                y = 0.5 * (y + d / y)
            dis = 1.0 / y
            dnv = 1.0 / d
            sq = y * _CK
            for q in range(_NQ):
                sl = pl.ds(q * _L, _L)
                abuf[i, sl] = dnv
                gbuf[0, i, sl] = gbuf[1, i, sl] * dis
                gbuf[1, i, sl] = sq
            return 0
        lax.fori_loop(0, _AC, _row, 0)
        pltpu.sync_copy(gbuf.at[0, pl.ds(0, _AC)],
                        z0.at[pl.ds(goff, _AC)])
        pltpu.sync_copy(abuf, dinv.at[pl.ds(goff, _AC)])
        pltpu.sync_copy(gbuf.at[1, pl.ds(0, _AC)],
                        sqc.at[pl.ds(goff, _AC)])


def _round_body(pidx, nfo, z_in, zsum_in, dinv, z_out, zsum_out,
                pbuf, src_i, dst_i, nbuf, gbuf, abuf, acc,
                p0, p1, g0, g1, s0, s1):
    c = lax.axis_index("c")
    s = lax.axis_index("s")
    cNH = c * _NH
    gbase = cNH + s * _RPC
    abase = s * _RPC
    gsems = (g0, g1)
    ssems = (s0, s1)

    for k in range(_NB):
        pltpu.sync_copy(z_in.at[pl.ds(gbase + k * _AC, _AC)], abuf)
        pltpu.sync_copy(abuf, acc.at[pl.ds(abase + k * _AC, _AC)])
    plsc.subcore_barrier()

    pltpu.sync_copy(nfo, nbuf)
    n0 = nbuf[pl.ds(0, _L)][0]
    g_lo = jnp.where(c == 0, 0, n0 // _GE)
    g_hi = jnp.where(c == 0, (n0 + _GE - 1) // _GE, _NGT)
    cnt = g_hi - g_lo
    t_lo = g_lo + s * cnt // _NS
    t_hi = g_lo + (s + 1) * cnt // _NS

    def _grp(g, _):
        pltpu.sync_copy(pidx.at[pl.ds(g * _GE, _GE)], pbuf.at[0])
        _unpack_group(pbuf, 0, src_i, dst_i, cNH, with_src=True)
        _pipeline(z_in, acc, src_i, dst_i, gbuf, gsems, ssems)
        return 0
    lax.fori_loop(t_lo, t_hi, _grp, 0)
    plsc.subcore_barrier()

    for k in range(_NB):
        goff = gbase + k * _AC
        pltpu.sync_copy(acc.at[pl.ds(abase + k * _AC, _AC)], abuf)
        pltpu.sync_copy(dinv.at[pl.ds(goff, _AC)],
                        gbuf.at[0, pl.ds(0, _AC)])
        pltpu.sync_copy(zsum_in.at[pl.ds(goff, _AC)],
                        gbuf.at[1, pl.ds(0, _AC)])

        def _row(i, _):
            for q in range(_NQ):
                sl = pl.ds(q * _L, _L)
                zv = abuf[i, sl] * gbuf[0, i, sl]
                abuf[i, sl] = zv
                gbuf[1, i, sl] = gbuf[1, i, sl] + zv
            return 0
        lax.fori_loop(0, _AC, _row, 0)
        pltpu.sync_copy(abuf, z_out.at[pl.ds(goff, _AC)])
        pltpu.sync_copy(gbuf.at[1, pl.ds(0, _AC)],
                        zsum_out.at[pl.ds(goff, _AC)])


def _sc_mesh():
    return plsc.VectorSubcoreMesh(core_axis_name="c", subcore_axis_name="s")


def _init_sc(xp, pidx):
    f = pl.kernel(
        _init_body,
        out_type=[jax.ShapeDtypeStruct((_NP, _D), jnp.float32)] * 3,
        mesh=_sc_mesh(),
        scratch_types=_scratch_types(),
    )
    return f(xp, pidx)


def _round_sc(pidx, nfo, z, zsum, dinv):
    f = pl.kernel(
        _round_body,
        out_type=[jax.ShapeDtypeStruct((_NP, _D), jnp.float32)] * 2,
        mesh=_sc_mesh(),
        scratch_types=_scratch_types(),
    )
    return f(pidx, nfo, z, zsum, dinv)


_MB = 1024


def _mlp_body(x_ref, zs_ref, sq_ref, wc_ref, bc_ref, w1_ref, b1_ref,
              w2_ref, b2_ref, o_ref):
    h = _ALPHA * x_ref[...] + zs_ref[...] * sq_ref[...]
    y = jnp.dot(h, wc_ref[...], preferred_element_type=jnp.float32)
    y = jnp.maximum(y + bc_ref[...], 0.0)
    y = jnp.dot(y, w1_ref[...], preferred_element_type=jnp.float32)
    y = jnp.maximum(y + b1_ref[...], 0.0)
    y = jnp.dot(y, w2_ref[...], preferred_element_type=jnp.float32)
    o_ref[...] = y + b2_ref[...]


def _mlp(xp, zsum, sqc, wc, bc, w1, b1, w2, b2):
    bspec = pl.BlockSpec((_MB, _D), lambda i: (i, 0))
    wspec = pl.BlockSpec((_D, _D), lambda i: (0, 0))
    vspec = pl.BlockSpec((1, _D), lambda i: (0, 0))
    return pl.pallas_call(
        _mlp_body,
        grid=(_NP // _MB,),
        in_specs=[bspec, bspec, bspec,
                  wspec, vspec, wspec, vspec, wspec, vspec],
        out_specs=bspec,
        out_shape=jax.ShapeDtypeStruct((_NP, _D), jnp.float32),
    )(xp, zsum, sqc, wc, bc.reshape(1, _D), w1, b1.reshape(1, _D),
      w2, b2.reshape(1, _D))


def kernel(x, edge_index, W_conv, b_conv, W1, b1, W2, b2):
    xp = jnp.pad(x, ((0, _NP - _N), (0, 0)))
    src = edge_index[0]
    dst = edge_index[1]
    packed = src | (dst << 16)
    m0 = dst < _NH
    c0 = jnp.cumsum(m0.astype(jnp.int32))
    n0 = c0[-1]
    c1 = jnp.cumsum(jnp.logical_not(m0).astype(jnp.int32))
    pos = jnp.where(m0, c0 - 1, n0 + c1 - 1)
    packed = jnp.zeros((_E,), jnp.int32).at[pos].set(packed)
    padn = _NS * _CH * _CB - _E
    pidx = jnp.concatenate(
        [packed, jnp.full((padn,), _NP << 16, jnp.int32)])
    nfo = jnp.full((_L,), n0, jnp.int32)

    z, dinv, sqc = _init_sc(xp, pidx)
    zsum = jnp.zeros((_NP, _D), jnp.float32)
    for _t in range(_K):
        z, zsum = _round_sc(pidx, nfo, z, zsum, dinv)
    out = _mlp(xp, zsum, sqc, W_conv, b_conv, W1, b1, W2, b2)
    return out[:_N]


# speedup vs baseline: 4.4047x; 1.0390x over previous
import jax
import jax.numpy as jnp
from jax import lax
from jax.experimental import pallas as pl
from jax.experimental.pallas import tpu as pltpu
from jax.experimental.pallas import tpu_sc as plsc

_N = 10000
_NP = 10240
_NH = _NP // 2
_E = 320000
_D = 128
_K = 16
_ALPHA = 0.6
_CK = (1.0 - _ALPHA) / _K
_NS = 16
_L = 16
_NQ = _D // _L
_CB = 128
_CH = 160
_GP = 16
_NG = _CH // _GP
_SS = 2
_SU = _GP // _SS
_GE = _GP * _CB
_NGT = _NS * _NG
_RPC = _NH // _NS
_AC = 64
_NB = _RPC // _AC
_DUMP = _NH


def _scratch_types():
    return [
        pltpu.VMEM((2, _GP * _CB), jnp.int32),
        pltpu.VMEM((_GP * _CB,), jnp.int32),
        pltpu.VMEM((_GP * _CB,), jnp.int32),
        pltpu.VMEM((_L,), jnp.int32),
        pltpu.VMEM((2, _SS * _CB, _D), jnp.float32),
        pltpu.VMEM((_AC, _D), jnp.float32),
        pltpu.VMEM_SHARED((_NH + 8, _D), jnp.float32),
    ] + [pltpu.SemaphoreType.DMA] * 6


def _unpack_group(pbuf, par, src_i, dst_i, cNH, with_src):
    for b in range(_GP):
        for j in range(_CB // _L):
            sl = pl.ds(b * _CB + j * _L, _L)
            v = pbuf[par, pl.ds(b * _CB + j * _L, _L)]
            d2 = (v >> 16) - cNH
            ok = (d2 >= 0) & (d2 < _NH)
            dst_i[sl] = jnp.where(ok, d2, _DUMP)
            if with_src:
                src_i[sl] = v & 0xFFFF


def _pipeline(z_in, acc, src_i, dst_i, gbuf, gsems, ssems):
    def _si(u):
        return src_i.at[pl.ds(_SS * _CB * u, _SS * _CB)]

    def _di(u):
        return dst_i.at[pl.ds(_SS * _CB * u, _SS * _CB)]

    pltpu.async_copy(z_in.at[_si(0)], gbuf.at[0], gsems[0])
    for u in range(_SU):
        r = u % 2
        pltpu.make_async_copy(z_in.at[_si(u)], gbuf.at[r], gsems[r]).wait()
        pltpu.async_copy(gbuf.at[r], acc.at[_di(u)], ssems[r], add=True)
        if u + 1 < _SU:
            if u >= 1:
                pltpu.make_async_copy(gbuf.at[1 - r], acc.at[_di(u - 1)],
                                      ssems[1 - r]).wait()
            pltpu.async_copy(z_in.at[_si(u + 1)], gbuf.at[1 - r],
                             gsems[1 - r])
    pltpu.make_async_copy(gbuf.at[0], acc.at[_di(_SU - 2)], ssems[0]).wait()
    pltpu.make_async_copy(gbuf.at[1], acc.at[_di(_SU - 1)], ssems[1]).wait()


def _init_body(xp, pidx, z0, dinv, sqc,
               pbuf, src_i, dst_i, nbuf, gbuf, abuf, acc,
               p0, p1, g0, g1, s0, s1):
    c = lax.axis_index("c")
    s = lax.axis_index("s")
    cNH = c * _NH
    gbase = cNH + s * _RPC
    abase = s * _RPC
    ssems = (s0, s1)

    def _zrow(i, _):
        for q in range(_NQ):
            abuf[i, pl.ds(q * _L, _L)] = jnp.zeros((_L,), jnp.float32)
        return 0
    lax.fori_loop(0, _AC, _zrow, 0)
    for k in range(_NB):
        pltpu.sync_copy(abuf, acc.at[pl.ds(abase + k * _AC, _AC)])
    plsc.subcore_barrier()

    def _orow(i, _):
        for q in range(_NQ):
            gbuf[0, i, pl.ds(q * _L, _L)] = jnp.ones((_L,), jnp.float32)
        return 0
    lax.fori_loop(0, _CB, _orow, 0)

    def _dgroup(g, _):
        pltpu.sync_copy(pidx.at[pl.ds(g * _GE, _GE)], pbuf.at[0])
        _unpack_group(pbuf, 0, src_i, dst_i, cNH, with_src=False)
        for b in range(_GP):
            r = b % 2
            ones = gbuf.at[0, pl.ds(0, _CB)]
            di = dst_i.at[pl.ds(b * _CB, _CB)]
            pltpu.async_copy(ones, acc.at[di], ssems[r], add=True)
            pltpu.make_async_copy(ones, acc.at[di], ssems[r]).wait()
        return 0
    lax.fori_loop(s * _NG, (s + 1) * _NG, _dgroup, 0)
    plsc.subcore_barrier()

    for k in range(_NB):
        goff = gbase + k * _AC
        pltpu.sync_copy(acc.at[pl.ds(abase + k * _AC, _AC)], abuf)
        pltpu.sync_copy(xp.at[pl.ds(goff, _AC)], gbuf.at[1, pl.ds(0, _AC)])

        def _row(i, _):
            d = abuf[i, pl.ds(0, _L)] + 1.0
            y = 0.5 * (d + 1.0)
            for _it in range(18):
                y = 0.5 * (y + d / y)
            dis = 1.0 / y
            dnv = 1.0 / d
            sq = y * _CK
            for q in range(_NQ):
                sl = pl.ds(q * _L, _L)
                abuf[i, sl] = dnv
                gbuf[0, i, sl] = gbuf[1, i, sl] * dis
                gbuf[1, i, sl] = sq
            return 0
        lax.fori_loop(0, _AC, _row, 0)
        pltpu.sync_copy(gbuf.at[0, pl.ds(0, _AC)],
                        z0.at[pl.ds(goff, _AC)])
        pltpu.sync_copy(abuf, dinv.at[pl.ds(goff, _AC)])
        pltpu.sync_copy(gbuf.at[1, pl.ds(0, _AC)],
                        sqc.at[pl.ds(goff, _AC)])


def _round_body(pidx, nfo, z_in, zsum_in, dinv, z_out, zsum_out,
                pbuf, src_i, dst_i, nbuf, gbuf, abuf, acc,
                p0, p1, g0, g1, s0, s1):
    c = lax.axis_index("c")
    s = lax.axis_index("s")
    cNH = c * _NH
    gbase = cNH + s * _RPC
    abase = s * _RPC
    gsems = (g0, g1)
    ssems = (s0, s1)

    for k in range(_NB):
        pltpu.sync_copy(z_in.at[pl.ds(gbase + k * _AC, _AC)], abuf)
        pltpu.sync_copy(abuf, acc.at[pl.ds(abase + k * _AC, _AC)])
    plsc.subcore_barrier()

    pltpu.sync_copy(nfo, nbuf)
    n0 = nbuf[pl.ds(0, _L)][0]
    g_lo = jnp.where(c == 0, 0, n0 // _GE)
    g_hi = jnp.where(c == 0, (n0 + _GE - 1) // _GE, _NGT)
    cnt = g_hi - g_lo
    t_lo = g_lo + s * cnt // _NS
    t_hi = g_lo + (s + 1) * cnt // _NS

    def _grp(g, _):
        pltpu.sync_copy(pidx.at[pl.ds(g * _GE, _GE)], pbuf.at[0])
        _unpack_group(pbuf, 0, src_i, dst_i, cNH, with_src=True)
        _pipeline(z_in, acc, src_i, dst_i, gbuf, gsems, ssems)
        return 0
    lax.fori_loop(t_lo, t_hi, _grp, 0)
    plsc.subcore_barrier()

    for k in range(_NB):
        goff = gbase + k * _AC
        pltpu.sync_copy(acc.at[pl.ds(abase + k * _AC, _AC)], abuf)
        pltpu.sync_copy(dinv.at[pl.ds(goff, _AC)],
                        gbuf.at[0, pl.ds(0, _AC)])
        pltpu.sync_copy(zsum_in.at[pl.ds(goff, _AC)],
                        gbuf.at[1, pl.ds(0, _AC)])

        def _row(i, _):
            for q in range(_NQ):
                sl = pl.ds(q * _L, _L)
                zv = abuf[i, sl] * gbuf[0, i, sl]
                abuf[i, sl] = zv
                gbuf[1, i, sl] = gbuf[1, i, sl] + zv
            return 0
        lax.fori_loop(0, _AC, _row, 0)
        pltpu.sync_copy(abuf, z_out.at[pl.ds(goff, _AC)])
        pltpu.sync_copy(gbuf.at[1, pl.ds(0, _AC)],
                        zsum_out.at[pl.ds(goff, _AC)])


def _sc_mesh():
    return plsc.VectorSubcoreMesh(core_axis_name="c", subcore_axis_name="s")


def _init_sc(xp, pidx):
    f = pl.kernel(
        _init_body,
        out_type=[jax.ShapeDtypeStruct((_NP, _D), jnp.float32)] * 3,
        mesh=_sc_mesh(),
        scratch_types=_scratch_types(),
    )
    return f(xp, pidx)


def _round_sc(pidx, nfo, z, zsum, dinv):
    f = pl.kernel(
        _round_body,
        out_type=[jax.ShapeDtypeStruct((_NP, _D), jnp.float32)] * 2,
        mesh=_sc_mesh(),
        scratch_types=_scratch_types(),
    )
    return f(pidx, nfo, z, zsum, dinv)


_MB = 1024


def _mlp_body(x_ref, zs_ref, sq_ref, wc_ref, bc_ref, w1_ref, b1_ref,
              w2_ref, b2_ref, o_ref):
    h = _ALPHA * x_ref[...] + zs_ref[...] * sq_ref[...]
    y = jnp.dot(h, wc_ref[...], preferred_element_type=jnp.float32)
    y = jnp.maximum(y + bc_ref[...], 0.0)
    y = jnp.dot(y, w1_ref[...], preferred_element_type=jnp.float32)
    y = jnp.maximum(y + b1_ref[...], 0.0)
    y = jnp.dot(y, w2_ref[...], preferred_element_type=jnp.float32)
    o_ref[...] = y + b2_ref[...]


def _mlp(xp, zsum, sqc, wc, bc, w1, b1, w2, b2):
    bspec = pl.BlockSpec((_MB, _D), lambda i: (i, 0))
    wspec = pl.BlockSpec((_D, _D), lambda i: (0, 0))
    vspec = pl.BlockSpec((1, _D), lambda i: (0, 0))
    return pl.pallas_call(
        _mlp_body,
        grid=(_NP // _MB,),
        in_specs=[bspec, bspec, bspec,
                  wspec, vspec, wspec, vspec, wspec, vspec],
        out_specs=bspec,
        out_shape=jax.ShapeDtypeStruct((_NP, _D), jnp.float32),
    )(xp, zsum, sqc, wc, bc.reshape(1, _D), w1, b1.reshape(1, _D),
      w2, b2.reshape(1, _D))


def kernel(x, edge_index, W_conv, b_conv, W1, b1, W2, b2):
    xp = jnp.pad(x, ((0, _NP - _N), (0, 0)))
    src = edge_index[0]
    dst = edge_index[1]
    packed = src | (dst << 16)
    m0 = dst < _NH
    n0 = jnp.sum(m0.astype(jnp.int32))
    key = (jnp.logical_not(m0).astype(jnp.int32) << 14) | src
    _, packed = lax.sort_key_val(key, packed)
    padn = _NS * _CH * _CB - _E
    pidx = jnp.concatenate(
        [packed, jnp.full((padn,), _NP << 16, jnp.int32)])
    nfo = jnp.full((_L,), n0, jnp.int32)

    z, dinv, sqc = _init_sc(xp, pidx)
    zsum = jnp.zeros((_NP, _D), jnp.float32)
    for _t in range(_K):
        z, zsum = _round_sc(pidx, nfo, z, zsum, dinv)
    out = _mlp(xp, zsum, sqc, W_conv, b_conv, W1, b1, W2, b2)
    return out[:_N]


# speedup vs baseline: 5.0758x; 1.1523x over previous
import jax
import jax.numpy as jnp
from jax import lax
from jax.experimental import pallas as pl
from jax.experimental.pallas import tpu as pltpu
from jax.experimental.pallas import tpu_sc as plsc

_N = 10000
_NP = 10240
_NH = _NP // 2
_E = 320000
_D = 128
_K = 16
_ALPHA = 0.6
_CK = (1.0 - _ALPHA) / _K
_NS = 16
_L = 16
_NQ = _D // _L
_CB = 128
_CH = 160
_GP = 16
_NG = _CH // _GP
_SS = 2
_SU = _GP // _SS
_GE = _GP * _CB
_NGT = _NS * _NG
_RPC = _NH // _NS
_AC = 64
_NB = _RPC // _AC
_DUMP = _NH


def _scratch_types():
    return [
        pltpu.VMEM((2, _GP * _CB), jnp.int32),
        pltpu.VMEM((_GP * _CB,), jnp.int32),
        pltpu.VMEM((_GP * _CB,), jnp.int32),
        pltpu.VMEM((_L,), jnp.int32),
        pltpu.VMEM((2, _SS * _CB, _D), jnp.float32),
        pltpu.VMEM((_AC, _D), jnp.float32),
        pltpu.VMEM_SHARED((_NH + 8, _D), jnp.float32),
    ] + [pltpu.SemaphoreType.DMA] * 6


def _unpack_group(pbuf, par, src_i, dst_i, cNH, with_src):
    for b in range(_GP):
        for j in range(_CB // _L):
            sl = pl.ds(b * _CB + j * _L, _L)
            v = pbuf[par, pl.ds(b * _CB + j * _L, _L)]
            d2 = (v >> 16) - cNH
            ok = (d2 >= 0) & (d2 < _NH)
            dst_i[sl] = jnp.where(ok, d2, _DUMP)
            if with_src:
                src_i[sl] = v & 0xFFFF


def _pipeline(z_in, acc, src_i, dst_i, gbuf, gsems, ssems):
    def _si(u):
        return src_i.at[pl.ds(_SS * _CB * u, _SS * _CB)]

    def _di(u):
        return dst_i.at[pl.ds(_SS * _CB * u, _SS * _CB)]

    pltpu.async_copy(z_in.at[_si(0)], gbuf.at[0], gsems[0])
    for u in range(_SU):
        r = u % 2
        pltpu.make_async_copy(z_in.at[_si(u)], gbuf.at[r], gsems[r]).wait()
        pltpu.async_copy(gbuf.at[r], acc.at[_di(u)], ssems[r], add=True)
        if u + 1 < _SU:
            if u >= 1:
                pltpu.make_async_copy(gbuf.at[1 - r], acc.at[_di(u - 1)],
                                      ssems[1 - r]).wait()
            pltpu.async_copy(z_in.at[_si(u + 1)], gbuf.at[1 - r],
                             gsems[1 - r])
    pltpu.make_async_copy(gbuf.at[0], acc.at[_di(_SU - 2)], ssems[0]).wait()
    pltpu.make_async_copy(gbuf.at[1], acc.at[_di(_SU - 1)], ssems[1]).wait()


def _init_body(xp, pidx, z0, dinv, sqc,
               pbuf, src_i, dst_i, nbuf, gbuf, abuf, acc,
               p0, p1, g0, g1, s0, s1):
    c = lax.axis_index("c")
    s = lax.axis_index("s")
    cNH = c * _NH
    gbase = cNH + s * _RPC
    abase = s * _RPC
    ssems = (s0, s1)

    def _zrow(i, _):
        for q in range(_NQ):
            abuf[i, pl.ds(q * _L, _L)] = jnp.zeros((_L,), jnp.float32)
        return 0
    lax.fori_loop(0, _AC, _zrow, 0)
    for k in range(_NB):
        pltpu.sync_copy(abuf, acc.at[pl.ds(abase + k * _AC, _AC)])
    plsc.subcore_barrier()

    def _orow(i, _):
        for q in range(_NQ):
            gbuf[0, i, pl.ds(q * _L, _L)] = jnp.ones((_L,), jnp.float32)
        return 0
    lax.fori_loop(0, _CB, _orow, 0)

    def _dgroup(g, _):
        pltpu.sync_copy(pidx.at[pl.ds(g * _GE, _GE)], pbuf.at[0])
        _unpack_group(pbuf, 0, src_i, dst_i, cNH, with_src=False)
        for b in range(_GP):
            r = b % 2
            ones = gbuf.at[0, pl.ds(0, _CB)]
            di = dst_i.at[pl.ds(b * _CB, _CB)]
            pltpu.async_copy(ones, acc.at[di], ssems[r], add=True)
            pltpu.make_async_copy(ones, acc.at[di], ssems[r]).wait()
        return 0
    lax.fori_loop(s * _NG, (s + 1) * _NG, _dgroup, 0)
    plsc.subcore_barrier()

    for k in range(_NB):
        goff = gbase + k * _AC
        pltpu.sync_copy(acc.at[pl.ds(abase + k * _AC, _AC)], abuf)
        pltpu.sync_copy(xp.at[pl.ds(goff, _AC)], gbuf.at[1, pl.ds(0, _AC)])

        def _row(i, _):
            d = abuf[i, pl.ds(0, _L)] + 1.0
            y = 0.5 * (d + 1.0)
            for _it in range(18):
                y = 0.5 * (y + d / y)
            dis = 1.0 / y
            dnv = 1.0 / d
            sq = y * _CK
            for q in range(_NQ):
                sl = pl.ds(q * _L, _L)
                abuf[i, sl] = dnv
                gbuf[0, i, sl] = gbuf[1, i, sl] * dis
                gbuf[1, i, sl] = sq
            return 0
        lax.fori_loop(0, _AC, _row, 0)
        pltpu.sync_copy(gbuf.at[0, pl.ds(0, _AC)],
                        z0.at[pl.ds(goff, _AC)])
        pltpu.sync_copy(abuf, dinv.at[pl.ds(goff, _AC)])
        pltpu.sync_copy(gbuf.at[1, pl.ds(0, _AC)],
                        sqc.at[pl.ds(goff, _AC)])


def _round_body(pidx, nfo, z_in, zsum_in, dinv, z_out, zsum_out,
                pbuf, src_i, dst_i, nbuf, gbuf, abuf, acc,
                p0, p1, g0, g1, s0, s1):
    c = lax.axis_index("c")
    s = lax.axis_index("s")
    cNH = c * _NH
    gbase = cNH + s * _RPC
    abase = s * _RPC
    gsems = (g0, g1)
    ssems = (s0, s1)

    for k in range(_NB):
        pltpu.sync_copy(z_in.at[pl.ds(gbase + k * _AC, _AC)], abuf)
        pltpu.sync_copy(abuf, acc.at[pl.ds(abase + k * _AC, _AC)])
    plsc.subcore_barrier()

    pltpu.sync_copy(nfo, nbuf)
    n0 = nbuf[pl.ds(0, _L)][0]
    g_lo = jnp.where(c == 0, 0, n0 // _GE)
    g_hi = jnp.where(c == 0, (n0 + _GE - 1) // _GE, _NGT)
    cnt = g_hi - g_lo
    t_lo = g_lo + s * cnt // _NS
    t_hi = g_lo + (s + 1) * cnt // _NS

    def _grp(g, _):
        pltpu.sync_copy(pidx.at[pl.ds(g * _GE, _GE)], pbuf.at[0])
        _unpack_group(pbuf, 0, src_i, dst_i, cNH, with_src=True)
        _pipeline(z_in, acc, src_i, dst_i, gbuf, gsems, ssems)
        return 0
    lax.fori_loop(t_lo, t_hi, _grp, 0)
    plsc.subcore_barrier()

    for k in range(_NB):
        goff = gbase + k * _AC
        pltpu.sync_copy(acc.at[pl.ds(abase + k * _AC, _AC)], abuf)
        pltpu.sync_copy(dinv.at[pl.ds(goff, _AC)],
                        gbuf.at[0, pl.ds(0, _AC)])
        pltpu.sync_copy(zsum_in.at[pl.ds(goff, _AC)],
                        gbuf.at[1, pl.ds(0, _AC)])

        def _row(i, _):
            for q in range(_NQ):
                sl = pl.ds(q * _L, _L)
                zv = abuf[i, sl] * gbuf[0, i, sl]
                abuf[i, sl] = zv
                gbuf[1, i, sl] = gbuf[1, i, sl] + zv
            return 0
        lax.fori_loop(0, _AC, _row, 0)
        pltpu.sync_copy(abuf, z_out.at[pl.ds(goff, _AC)])
        pltpu.sync_copy(gbuf.at[1, pl.ds(0, _AC)],
                        zsum_out.at[pl.ds(goff, _AC)])


def _sc_mesh():
    return plsc.VectorSubcoreMesh(core_axis_name="c", subcore_axis_name="s")


def _init_sc(xp, pidx):
    f = pl.kernel(
        _init_body,
        out_type=[jax.ShapeDtypeStruct((_NP, _D), jnp.float32)] * 3,
        mesh=_sc_mesh(),
        scratch_types=_scratch_types(),
    )
    return f(xp, pidx)


def _round_sc(pidx, nfo, z, zsum, dinv):
    f = pl.kernel(
        _round_body,
        out_type=[jax.ShapeDtypeStruct((_NP, _D), jnp.float32)] * 2,
        mesh=_sc_mesh(),
        scratch_types=_scratch_types(),
    )
    return f(pidx, nfo, z, zsum, dinv)


_MB = 1024


def _mlp_body(x_ref, zs_ref, sq_ref, wc_ref, bc_ref, w1_ref, b1_ref,
              w2_ref, b2_ref, o_ref):
    h = _ALPHA * x_ref[...] + zs_ref[...] * sq_ref[...]
    y = jnp.dot(h, wc_ref[...], preferred_element_type=jnp.float32)
    y = jnp.maximum(y + bc_ref[...], 0.0)
    y = jnp.dot(y, w1_ref[...], preferred_element_type=jnp.float32)
    y = jnp.maximum(y + b1_ref[...], 0.0)
    y = jnp.dot(y, w2_ref[...], preferred_element_type=jnp.float32)
    o_ref[...] = y + b2_ref[...]


def _mlp(xp, zsum, sqc, wc, bc, w1, b1, w2, b2):
    bspec = pl.BlockSpec((_MB, _D), lambda i: (i, 0))
    wspec = pl.BlockSpec((_D, _D), lambda i: (0, 0))
    vspec = pl.BlockSpec((1, _D), lambda i: (0, 0))
    return pl.pallas_call(
        _mlp_body,
        grid=(_NP // _MB,),
        in_specs=[bspec, bspec, bspec,
                  wspec, vspec, wspec, vspec, wspec, vspec],
        out_specs=bspec,
        out_shape=jax.ShapeDtypeStruct((_NP, _D), jnp.float32),
    )(xp, zsum, sqc, wc, bc.reshape(1, _D), w1, b1.reshape(1, _D),
      w2, b2.reshape(1, _D))


def kernel(x, edge_index, W_conv, b_conv, W1, b1, W2, b2):
    xp = jnp.pad(x, ((0, _NP - _N), (0, 0)))
    src = edge_index[0]
    dst = edge_index[1]
    packed = src | (dst << 16)
    m0 = dst < _NH
    n0 = jnp.sum(m0.astype(jnp.int32))
    packed = jnp.sort(packed)
    padn = _NS * _CH * _CB - _E
    pidx = jnp.concatenate(
        [packed, jnp.full((padn,), _NP << 16, jnp.int32)])
    nfo = jnp.full((_L,), n0, jnp.int32)

    z, dinv, sqc = _init_sc(xp, pidx)
    zsum = jnp.zeros((_NP, _D), jnp.float32)
    for _t in range(_K):
        z, zsum = _round_sc(pidx, nfo, z, zsum, dinv)
    out = _mlp(xp, zsum, sqc, W_conv, b_conv, W1, b1, W2, b2)
    return out[:_N]
